# baseline (speedup 1.0000x reference)
import functools

import jax
import jax.numpy as jnp
from jax import lax
from jax.experimental import pallas as pl
from jax.experimental.pallas import tpu as pltpu
from jax.experimental.pallas import tpu_sc as plsc

_NC = 2
_NS = 16
_LANES = 16
_NW = _NC * _NS
_K = 128
_SS = 4
_NIB = 8
_R = 2
_D = 1


def _row_block_spec(rb, c):
    return pl.BlockSpec((rb, c), lambda i: (i, 0))


def _matmul(x, weight):
    n, c = x.shape
    rb = 2000
    def body(x_ref, w_ref, o_ref):
        o_ref[...] = jnp.dot(x_ref[...], w_ref[...],
                             preferred_element_type=jnp.float32)
    return pl.pallas_call(
        body,
        grid=(n // rb,),
        in_specs=[_row_block_spec(rb, c), pl.BlockSpec((c, c), lambda i: (0, 0))],
        out_specs=_row_block_spec(rb, c),
        out_shape=jax.ShapeDtypeStruct((n, c), jnp.float32),
    )(x, weight)


def _combine(a, b, x=None):
    n, c = a.shape
    rb = 2000
    if x is None:
        def body(a_ref, b_ref, o_ref):
            o_ref[...] = a_ref[...] + b_ref[...]
        args = (a, b)
        specs = [_row_block_spec(rb, c)] * 2
    else:
        def body(a_ref, b_ref, x_ref, o_ref):
            o_ref[...] = x_ref[...] + (a_ref[...] + b_ref[...])
        args = (a, b, x)
        specs = [_row_block_spec(rb, c)] * 3
    return pl.pallas_call(
        body,
        grid=(n // rb,),
        in_specs=specs,
        out_specs=_row_block_spec(rb, c),
        out_shape=jax.ShapeDtypeStruct((n, c), jnp.float32),
    )(*args)


def _spmm_sc(src, col_pad, row_pad, val_pad):
    n_src, c = src.shape
    n_acc = -(-n_src // (8 * _NS)) * (8 * _NS)
    ep = col_pad.shape[0]
    epw = ep // _NW
    nchunks = epw // _K
    assert nchunks % _SS == 0
    rpt = n_acc // _NS
    full, rem = divmod(rpt, _K)
    mesh = plsc.VectorSubcoreMesh(core_axis_name="c", subcore_axis_name="s")

    @functools.partial(
        pl.kernel,
        out_type=jax.ShapeDtypeStruct((2 * n_acc, c), jnp.float32),
        mesh=mesh,
        scratch_types=(
            [pltpu.VMEM((_NIB * _SS * _K,), jnp.int32),
             pltpu.VMEM((_NIB * _SS * _K,), jnp.float32),
             pltpu.VMEM((_NIB * _SS * _K,), jnp.int32),
             pltpu.VMEM((_R, _K, c), jnp.float32),
             pltpu.VMEM_SHARED((n_acc, c), jnp.float32),
             pltpu.SemaphoreType.DMA, pltpu.SemaphoreType.DMA]
            + [pltpu.SemaphoreType.DMA for _ in range(2 * _R)]
        ),
    )
    def spmm(src_hbm, col_hbm, row_hbm, val_hbm, out_hbm, *refs):
        colv, valv, rowm = refs[0], refs[1], refs[2]
        buf, acc = refs[3], refs[4]
        isem, vsem = refs[5], refs[6]
        gsem = refs[7:7 + _R]
        ssem = refs[7 + _R:7 + 2 * _R]
        cid = lax.axis_index("c")
        sid = lax.axis_index("s")
        wid = sid * _NC + cid

        def zero_row(i, carry):
            for j in range(c // _LANES):
                buf[0, i, pl.ds(j * _LANES, _LANES)] = jnp.zeros((_LANES,), jnp.float32)
            return carry
        lax.fori_loop(0, _K, zero_row, 0)
        zbuf = buf.at[0]
        for i in range(full):
            pltpu.sync_copy(zbuf, acc.at[pl.ds(sid * rpt + i * _K, _K)])
        if rem:
            pltpu.sync_copy(zbuf.at[pl.ds(0, rem)],
                            acc.at[pl.ds(sid * rpt + full * _K, rem)])
        plsc.subcore_barrier()

        def scale(buf_b, voff):
            def group(kg, carry):
                vv = valv[pl.ds(voff + kg * _LANES, _LANES)]
                for j in range(_LANES):
                    k = kg * _LANES + j
                    s = vv[j]
                    for m in range(c // _LANES):
                        sl = pl.ds(m * _LANES, _LANES)
                        buf_b[k, sl] = buf_b[k, sl] * s
                return carry
            lax.fori_loop(0, _K // _LANES, group, 0)

        def load_idx(base, ne):
            ih = pltpu.async_copy(row_hbm.at[pl.ds(base, ne)],
                                  rowm.at[pl.ds(0, ne)], isem)
            vh = pltpu.async_copy(val_hbm.at[pl.ds(base, ne)],
                                  valv.at[pl.ds(0, ne)], vsem)
            pltpu.sync_copy(col_hbm.at[pl.ds(base, ne)],
                            colv.at[pl.ds(0, ne)])
            vh.wait()
            ih.wait()

        def process(off):
            def fire_gather(b):
                return pltpu.async_copy(
                    src_hbm.at[colv.at[pl.ds(off + b * _K, _K)]],
                    buf.at[b % _R], gsem[b % _R])
            gh = [None] * _SS
            sh = [None] * _SS
            for b in range(_D):
                gh[b] = fire_gather(b)
            for b in range(_SS):
                nxt = b + _D
                if nxt < _SS:
                    prev = nxt - _R
                    if prev >= 0:
                        sh[prev].wait()
                        sh[prev] = None
                    gh[nxt] = fire_gather(nxt)
                gh[b].wait()
                scale(buf.at[b % _R], off + b * _K)
                sh[b] = pltpu.async_copy(
                    buf.at[b % _R],
                    acc.at[rowm.at[pl.ds(off + b * _K, _K)]],
                    ssem[b % _R], add=True)
            for h in sh:
                if h is not None:
                    h.wait()

        g4 = _NIB * _SS * _K
        ngrp, tail_blocks = divmod(nchunks // _SS, _NIB)

        def grp(s, carry):
            load_idx(wid * epw + s * g4, g4)

            def blk(ib, c2):
                process(ib * (_SS * _K))
                return c2
            lax.fori_loop(0, _NIB, blk, 0)
            return carry
        lax.fori_loop(0, ngrp, grp, 0)
        if tail_blocks:
            load_idx(wid * epw + ngrp * g4, tail_blocks * _SS * _K)

            def blk2(ib, c2):
                process(ib * (_SS * _K))
                return c2
            lax.fori_loop(0, tail_blocks, blk2, 0)
        plsc.subcore_barrier()

        for i in range(full):
            pltpu.sync_copy(acc.at[pl.ds(sid * rpt + i * _K, _K)],
                            out_hbm.at[pl.ds(cid * n_acc + sid * rpt + i * _K, _K)])
        if rem:
            pltpu.sync_copy(acc.at[pl.ds(sid * rpt + full * _K, rem)],
                            out_hbm.at[pl.ds(cid * n_acc + sid * rpt + full * _K, rem)])

    return spmm(src, col_pad, row_pad, val_pad)


def kernel(x, weight, lap_val, lap_row, lap_col):
    n, c = x.shape
    nnz = lap_row.shape[0]
    n_acc = -(-n // (8 * _NS)) * (8 * _NS)
    pad_rows = n_acc - n
    ep = -(-nnz // (_NW * _K * _SS)) * (_NW * _K * _SS)
    padn = ep - nnz
    spread = jnp.arange(padn, dtype=jnp.int32) % 8
    col_pad = jnp.concatenate([lap_col.astype(jnp.int32), spread * 8])
    row_pad = jnp.concatenate([lap_row.astype(jnp.int32),
                               n + (jnp.arange(padn, dtype=jnp.int32) % pad_rows)])
    val_pad = jnp.concatenate([lap_val, jnp.zeros((padn,), jnp.float32)])

    z = _matmul(x, weight)
    w_parts = _spmm_sc(z, col_pad, row_pad, val_pad)
    w = _combine(w_parts[:n], w_parts[n_acc:n_acc + n])
    v_parts = _spmm_sc(w, col_pad, row_pad, val_pad)
    return _combine(v_parts[:n], v_parts[n_acc:n_acc + n], x=x)


# speedup vs baseline: 1.0798x; 1.0798x over previous
import functools

import jax
import jax.numpy as jnp
from jax import lax
from jax.experimental import pallas as pl
from jax.experimental.pallas import tpu as pltpu
from jax.experimental.pallas import tpu_sc as plsc

_NC = 2
_NS = 16
_LANES = 16
_NW = _NC * _NS
_K = 64
_SS = 8
_NIB = 4
_R = 5
_D = 3


def _row_block_spec(rb, c):
    return pl.BlockSpec((rb, c), lambda i: (i, 0))


def _matmul(x, weight):
    n, c = x.shape
    rb = 2000
    def body(x_ref, w_ref, o_ref):
        o_ref[...] = jnp.dot(x_ref[...], w_ref[...],
                             preferred_element_type=jnp.float32)
    return pl.pallas_call(
        body,
        grid=(n // rb,),
        in_specs=[_row_block_spec(rb, c), pl.BlockSpec((c, c), lambda i: (0, 0))],
        out_specs=_row_block_spec(rb, c),
        out_shape=jax.ShapeDtypeStruct((n, c), jnp.float32),
    )(x, weight)


def _combine(a, b, x=None):
    n, c = a.shape
    rb = 2000
    if x is None:
        def body(a_ref, b_ref, o_ref):
            o_ref[...] = a_ref[...] + b_ref[...]
        args = (a, b)
        specs = [_row_block_spec(rb, c)] * 2
    else:
        def body(a_ref, b_ref, x_ref, o_ref):
            o_ref[...] = x_ref[...] + (a_ref[...] + b_ref[...])
        args = (a, b, x)
        specs = [_row_block_spec(rb, c)] * 3
    return pl.pallas_call(
        body,
        grid=(n // rb,),
        in_specs=specs,
        out_specs=_row_block_spec(rb, c),
        out_shape=jax.ShapeDtypeStruct((n, c), jnp.float32),
    )(*args)


def _spmm_sc(src, col_pad, row_pad, val_pad):
    n_src, c = src.shape
    n_acc = -(-n_src // (8 * _NS)) * (8 * _NS)
    ep = col_pad.shape[0]
    epw = ep // _NW
    nchunks = epw // _K
    assert nchunks % _SS == 0
    rpt = n_acc // _NS
    full, rem = divmod(rpt, _K)
    mesh = plsc.VectorSubcoreMesh(core_axis_name="c", subcore_axis_name="s")

    @functools.partial(
        pl.kernel,
        out_type=jax.ShapeDtypeStruct((2 * n_acc, c), jnp.float32),
        mesh=mesh,
        scratch_types=(
            [pltpu.VMEM((_NIB * _SS * _K,), jnp.int32),
             pltpu.VMEM((_NIB * _SS * _K,), jnp.float32),
             pltpu.VMEM((_NIB * _SS * _K,), jnp.int32),
             pltpu.VMEM((_R, _K, c), jnp.float32),
             pltpu.VMEM_SHARED((n_acc, c), jnp.float32),
             pltpu.SemaphoreType.DMA, pltpu.SemaphoreType.DMA]
            + [pltpu.SemaphoreType.DMA for _ in range(2 * _R)]
        ),
    )
    def spmm(src_hbm, col_hbm, row_hbm, val_hbm, out_hbm, *refs):
        colv, valv, rowm = refs[0], refs[1], refs[2]
        buf, acc = refs[3], refs[4]
        isem, vsem = refs[5], refs[6]
        gsem = refs[7:7 + _R]
        ssem = refs[7 + _R:7 + 2 * _R]
        cid = lax.axis_index("c")
        sid = lax.axis_index("s")
        wid = sid * _NC + cid

        def zero_row(i, carry):
            for j in range(c // _LANES):
                buf[0, i, pl.ds(j * _LANES, _LANES)] = jnp.zeros((_LANES,), jnp.float32)
            return carry
        lax.fori_loop(0, _K, zero_row, 0)
        zbuf = buf.at[0]
        for i in range(full):
            pltpu.sync_copy(zbuf, acc.at[pl.ds(sid * rpt + i * _K, _K)])
        if rem:
            pltpu.sync_copy(zbuf.at[pl.ds(0, rem)],
                            acc.at[pl.ds(sid * rpt + full * _K, rem)])
        plsc.subcore_barrier()

        def scale(buf_b, voff):
            def group(kg, carry):
                vv = valv[pl.ds(voff + kg * _LANES, _LANES)]
                for j in range(_LANES):
                    k = kg * _LANES + j
                    s = vv[j]
                    for m in range(c // _LANES):
                        sl = pl.ds(m * _LANES, _LANES)
                        buf_b[k, sl] = buf_b[k, sl] * s
                return carry
            lax.fori_loop(0, _K // _LANES, group, 0)

        def load_idx(base, ne):
            ih = pltpu.async_copy(row_hbm.at[pl.ds(base, ne)],
                                  rowm.at[pl.ds(0, ne)], isem)
            vh = pltpu.async_copy(val_hbm.at[pl.ds(base, ne)],
                                  valv.at[pl.ds(0, ne)], vsem)
            pltpu.sync_copy(col_hbm.at[pl.ds(base, ne)],
                            colv.at[pl.ds(0, ne)])
            vh.wait()
            ih.wait()

        def process(off):
            def fire_gather(b):
                return pltpu.async_copy(
                    src_hbm.at[colv.at[pl.ds(off + b * _K, _K)]],
                    buf.at[b % _R], gsem[b % _R])
            gh = [None] * _SS
            sh = [None] * _SS
            for b in range(_D):
                gh[b] = fire_gather(b)
            for b in range(_SS):
                nxt = b + _D
                if nxt < _SS:
                    prev = nxt - _R
                    if prev >= 0:
                        sh[prev].wait()
                        sh[prev] = None
                    gh[nxt] = fire_gather(nxt)
                gh[b].wait()
                scale(buf.at[b % _R], off + b * _K)
                sh[b] = pltpu.async_copy(
                    buf.at[b % _R],
                    acc.at[rowm.at[pl.ds(off + b * _K, _K)]],
                    ssem[b % _R], add=True)
            for h in sh:
                if h is not None:
                    h.wait()

        g4 = _NIB * _SS * _K
        ngrp, tail_blocks = divmod(nchunks // _SS, _NIB)

        def grp(s, carry):
            load_idx(wid * epw + s * g4, g4)

            def blk(ib, c2):
                process(ib * (_SS * _K))
                return c2
            lax.fori_loop(0, _NIB, blk, 0)
            return carry
        lax.fori_loop(0, ngrp, grp, 0)
        if tail_blocks:
            load_idx(wid * epw + ngrp * g4, tail_blocks * _SS * _K)

            def blk2(ib, c2):
                process(ib * (_SS * _K))
                return c2
            lax.fori_loop(0, tail_blocks, blk2, 0)
        plsc.subcore_barrier()

        for i in range(full):
            pltpu.sync_copy(acc.at[pl.ds(sid * rpt + i * _K, _K)],
                            out_hbm.at[pl.ds(cid * n_acc + sid * rpt + i * _K, _K)])
        if rem:
            pltpu.sync_copy(acc.at[pl.ds(sid * rpt + full * _K, rem)],
                            out_hbm.at[pl.ds(cid * n_acc + sid * rpt + full * _K, rem)])

    return spmm(src, col_pad, row_pad, val_pad)


def kernel(x, weight, lap_val, lap_row, lap_col):
    n, c = x.shape
    nnz = lap_row.shape[0]
    n_acc = -(-n // (8 * _NS)) * (8 * _NS)
    pad_rows = n_acc - n
    ep = -(-nnz // (_NW * _K * _SS)) * (_NW * _K * _SS)
    padn = ep - nnz
    spread = jnp.arange(padn, dtype=jnp.int32) % 8
    col_pad = jnp.concatenate([lap_col.astype(jnp.int32), spread * 8])
    row_pad = jnp.concatenate([lap_row.astype(jnp.int32),
                               n + (jnp.arange(padn, dtype=jnp.int32) % pad_rows)])
    val_pad = jnp.concatenate([lap_val, jnp.zeros((padn,), jnp.float32)])

    z = _matmul(x, weight)
    w_parts = _spmm_sc(z, col_pad, row_pad, val_pad)
    w = _combine(w_parts[:n], w_parts[n_acc:n_acc + n])
    v_parts = _spmm_sc(w, col_pad, row_pad, val_pad)
    return _combine(v_parts[:n], v_parts[n_acc:n_acc + n], x=x)
